# baseline (speedup 1.0000x reference)
import functools

import jax
import jax.numpy as jnp
from jax import lax
from jax.experimental import pallas as pl
from jax.experimental.pallas import tpu as pltpu
from jax.experimental.pallas import tpu_sc as plsc

N_VOX = 50000
N_VOXP = 50048
K_VOL = 27
PAIRS = 25000
C = 64

HALF = N_VOX // 2
TRASH = HALF
ACC_ROWS = HALF + 8
CHUNK = 128
N_SUB = 16
N_PAIRS = K_VOL * PAIRS
N_PAIRS_PAD = 675840
CHUNKS_PER_TILE = N_PAIRS_PAD // CHUNK // N_SUB
SEG = 22
N_SEG = CHUNKS_PER_TILE // SEG
INIT_ROWS_PER_TILE = ACC_ROWS // N_SUB
OUT_FULL_CHUNKS = HALF // CHUNK
OUT_TAIL = HALF - OUT_FULL_CHUNKS * CHUNK

MM_BLOCK = 3128


def _rb16(y):
    u = jax.lax.bitcast_convert_type(y, jnp.int32)
    return (u + 0x7FFF + ((u >> 16) & 1)) >> 16


def _mm_body(x_ref, w_ref, y_ref):
    w = w_ref[0]
    packs = []
    for q in range(4):
        y = jnp.dot(
            x_ref[:, q * C:(q + 1) * C], w,
            preferred_element_type=jnp.float32,
        )
        lo = _rb16(y[:, :C // 2]) & 0xFFFF
        hi = _rb16(y[:, C // 2:]) << 16
        packs.append(lo | hi)
    y_ref[...] = jax.lax.bitcast_convert_type(
        jnp.concatenate(packs, axis=1), jnp.float32
    )


def _dense_transform(in4, weights):
    nb = (N_VOXP // 4) // MM_BLOCK
    return pl.pallas_call(
        _mm_body,
        grid=(nb, K_VOL),
        in_specs=[
            pl.BlockSpec((MM_BLOCK, 4 * C), lambda j, k: (j, 0)),
            pl.BlockSpec((1, C, C), lambda j, k: (k, 0, 0)),
        ],
        out_specs=pl.BlockSpec(
            (MM_BLOCK, 2 * C), lambda j, k: (k * nb + j, 0)
        ),
        out_shape=jax.ShapeDtypeStruct((K_VOL * N_VOXP // 4, 2 * C), jnp.float32),
    )(in4, weights)


def _sc_scatter(y_flat, gidx2, oloc3, binit):
    mesh = plsc.VectorSubcoreMesh(core_axis_name="c", subcore_axis_name="s")

    @functools.partial(
        pl.kernel,
        mesh=mesh,
        compiler_params=pltpu.CompilerParams(use_tc_tiling_on_sc=False),
        out_type=jax.ShapeDtypeStruct((N_VOX, C), jnp.bfloat16),
        scratch_types=[
            pltpu.VMEM((SEG, CHUNK), jnp.int32),
            pltpu.VMEM((SEG, CHUNK), jnp.int32),
            pltpu.VMEM((SEG, CHUNK), jnp.int32),
            pltpu.VMEM((SEG, CHUNK), jnp.int32),
            pltpu.VMEM((CHUNK, C), jnp.bfloat16),
            pltpu.VMEM((CHUNK, C), jnp.bfloat16),
            pltpu.VMEM_SHARED((ACC_ROWS, C), jnp.bfloat16),
            pltpu.SemaphoreType.DMA,
            pltpu.SemaphoreType.DMA,
            pltpu.SemaphoreType.DMA,
            pltpu.SemaphoreType.DMA,
            pltpu.SemaphoreType.DMA,
        ],
    )
    def body(y_hbm, gidx_hbm, oloc_hbm, binit_hbm, out_hbm,
             gs0, gs1, os0, os1, rows0, rows1, acc,
             sg0, sg1, ss0, ss1, sem_out):
        cid = lax.axis_index("c")
        sid = lax.axis_index("s")
        G = (gs0, gs1)
        O = (os0, os1)
        rows_b = (rows0, rows1)
        sg = (sg0, sg1)
        ss = (ss0, ss1)

        init_base = sid * INIT_ROWS_PER_TILE
        pltpu.sync_copy(
            binit_hbm.at[pl.ds(init_base, INIT_ROWS_PER_TILE)],
            acc.at[pl.ds(init_base, INIT_ROWS_PER_TILE)],
        )
        plsc.subcore_barrier()

        tile_row0 = sid * CHUNKS_PER_TILE

        def gload(s, p):
            srow = tile_row0 + s * SEG
            pltpu.sync_copy(gidx_hbm.at[pl.ds(srow, SEG)], G[p])
            pltpu.sync_copy(oloc_hbm.at[cid, pl.ds(srow, SEG)], O[p])

        def fire_g(b, idx_row):
            pltpu.make_async_copy(
                y_hbm.at[idx_row], rows_b[b], sg[b]
            ).start()

        def wait_g(b, idx_row):
            pltpu.make_async_copy(
                y_hbm.at[idx_row], rows_b[b], sg[b]
            ).wait()

        def fire_s(b, idx_row):
            pltpu.make_async_copy(
                rows_b[b], acc.at[idx_row], ss[b]
            ).start(add=True)

        def wait_s(b, idx_row):
            pltpu.make_async_copy(
                rows_b[b], acc.at[idx_row], ss[b]
            ).wait()

        def sub_iter(b, p, j, next_ref, first):
            if not first:
                wait_s(1 - b, O[p].at[j])
            if next_ref is not None:
                fire_g(1 - b, next_ref)
            wait_g(b, G[p].at[j])
            fire_s(b, O[p].at[j])

        gload(0, 0)
        fire_g(0, G[0].at[0])

        for s in range(N_SEG):
            p = s % 2
            sub_iter(0, p, 0, G[p].at[1], first=(s == 0))
            if s + 1 < N_SEG:
                gload(s + 1, 1 - p)
            sub_iter(1, p, 1, G[p].at[2], first=False)

            def mid(jj, _):
                sub_iter(0, p, 2 * jj, G[p].at[2 * jj + 1], first=False)
                sub_iter(1, p, 2 * jj + 1, G[p].at[2 * jj + 2], first=False)
                return 0

            lax.fori_loop(1, SEG // 2 - 1, mid, 0)
            sub_iter(0, p, SEG - 2, G[p].at[SEG - 1], first=False)
            last_next = None if s + 1 == N_SEG else G[1 - p].at[0]
            sub_iter(1, p, SEG - 1, last_next, first=False)

        wait_s(1, O[(N_SEG - 1) % 2].at[SEG - 1])
        plsc.subcore_barrier()

        row_base = cid * HALF

        def out_descs():
            for i in range(OUT_FULL_CHUNKS // N_SUB + 1):
                chunk = i * N_SUB + sid
                off = chunk * CHUNK
                full = pltpu.make_async_copy(
                    acc.at[pl.ds(off, CHUNK)],
                    out_hbm.at[pl.ds(row_base + off, CHUNK)],
                    sem_out,
                )
                tail = pltpu.make_async_copy(
                    acc.at[pl.ds(OUT_FULL_CHUNKS * CHUNK, OUT_TAIL)],
                    out_hbm.at[
                        pl.ds(row_base + OUT_FULL_CHUNKS * CHUNK, OUT_TAIL)
                    ],
                    sem_out,
                )
                yield chunk, full, tail

        for chunk, full, tail in out_descs():
            @pl.when(chunk < OUT_FULL_CHUNKS)
            def _():
                full.start()

            @pl.when(chunk == OUT_FULL_CHUNKS)
            def _():
                tail.start()

        for chunk, full, tail in out_descs():
            @pl.when(chunk < OUT_FULL_CHUNKS)
            def _():
                full.wait()

            @pl.when(chunk == OUT_FULL_CHUNKS)
            def _():
                tail.wait()

    return body(y_flat, gidx2, oloc3, binit)


def kernel(in_feats, imap, omap, kernel, bias):
    imap = imap.astype(jnp.int32)
    omap = omap.astype(jnp.int32)

    inp = jnp.pad(in_feats, ((0, N_VOXP - N_VOX), (0, 0)))
    in4 = inp.reshape(N_VOXP // 4, 4 * C)
    w_perm = jnp.concatenate([kernel[:, :, 0::2], kernel[:, :, 1::2]], axis=2)
    y_packed = _dense_transform(in4, w_perm)
    y16 = jax.lax.bitcast_convert_type(y_packed, jnp.bfloat16)
    y_flat = y16.reshape(K_VOL * N_VOXP, C)

    k_off = (jnp.arange(K_VOL, dtype=jnp.int32) * N_VOXP)[:, None]
    gidx = (imap + k_off).reshape(-1)
    pad = N_PAIRS_PAD - N_PAIRS
    gidx = jnp.concatenate([gidx, jnp.zeros((pad,), jnp.int32)])
    omap_flat = jnp.concatenate(
        [omap.reshape(-1), jnp.full((pad,), N_VOX, jnp.int32)]
    )
    oloc0 = jnp.where(omap_flat < HALF, omap_flat, TRASH)
    oloc1 = jnp.where(omap_flat >= HALF, omap_flat - HALF, TRASH)
    gidx2 = gidx.reshape(N_SUB * CHUNKS_PER_TILE, CHUNK)
    oloc3 = jnp.stack(
        [oloc0.reshape(N_SUB * CHUNKS_PER_TILE, CHUNK),
         oloc1.reshape(N_SUB * CHUNKS_PER_TILE, CHUNK)]
    )
    binit = jnp.broadcast_to(bias.astype(jnp.bfloat16), (ACC_ROWS, C))
    out16 = _sc_scatter(y_flat, gidx2, oloc3, binit)
    return out16.astype(jnp.float32)


# speedup vs baseline: 76.6429x; 76.6429x over previous
import functools

import jax
import jax.numpy as jnp
from jax import lax
from jax.experimental import pallas as pl
from jax.experimental.pallas import tpu as pltpu
from jax.experimental.pallas import tpu_sc as plsc

N_VOX = 50000
K_VOL = 27
PAIRS = 25000
C = 64

HALF = N_VOX // 2
TRASH = HALF
ACC_ROWS = HALF + 8
CHUNK = 128
N_SUB = 16
N_PAIRS = K_VOL * PAIRS
N_PAIRS_PAD = 675840
CHUNKS_PER_TILE = N_PAIRS_PAD // CHUNK // N_SUB
INIT_ROWS_PER_TILE = ACC_ROWS // N_SUB
OUT_FULL_CHUNKS = HALF // CHUNK
OUT_TAIL = HALF - OUT_FULL_CHUNKS * CHUNK

MM_BLOCK = 5000


def _mm_body(x_ref, w_ref, y_ref):
    w = w_ref[0]
    y_ref[:, :C] = jnp.dot(
        x_ref[:, :C], w, preferred_element_type=jnp.float32
    )
    y_ref[:, C:] = jnp.dot(
        x_ref[:, C:], w, preferred_element_type=jnp.float32
    )


def _dense_transform(in2, weights):
    nb = (N_VOX // 2) // MM_BLOCK
    return pl.pallas_call(
        _mm_body,
        grid=(nb, K_VOL),
        in_specs=[
            pl.BlockSpec((MM_BLOCK, 2 * C), lambda j, k: (j, 0)),
            pl.BlockSpec((1, C, C), lambda j, k: (k, 0, 0)),
        ],
        out_specs=pl.BlockSpec((MM_BLOCK, 2 * C), lambda j, k: (k * nb + j, 0)),
        out_shape=jax.ShapeDtypeStruct((K_VOL * N_VOX // 2, 2 * C), jnp.float32),
    )(in2, weights)


def _sc_scatter(y_flat, gidx, omap_flat, bias_row):
    mesh = plsc.VectorSubcoreMesh(core_axis_name="c", subcore_axis_name="s")

    @functools.partial(
        pl.kernel,
        mesh=mesh,
        compiler_params=pltpu.CompilerParams(use_tc_tiling_on_sc=False),
        out_type=jax.ShapeDtypeStruct((N_VOX, C), jnp.float32),
        scratch_types=[
            pltpu.VMEM((CHUNK,), jnp.int32),
            pltpu.VMEM((CHUNK,), jnp.int32),
            pltpu.VMEM((CHUNK,), jnp.int32),
            pltpu.VMEM((CHUNK,), jnp.int32),
            pltpu.VMEM((CHUNK,), jnp.int32),
            pltpu.VMEM((CHUNK, C), jnp.float32),
            pltpu.VMEM((CHUNK, C), jnp.float32),
            pltpu.VMEM((CHUNK, C), jnp.float32),
            pltpu.VMEM_SHARED((ACC_ROWS, C), jnp.float32),
            pltpu.SemaphoreType.DMA,
            pltpu.SemaphoreType.DMA,
        ],
    )
    def body(y_hbm, gidx_hbm, omap_hbm, bias_hbm, out_hbm,
             gidx0, gidx1, omap0, omap1, idx_v, rows0, rows1, bias_v, acc,
             sem0, sem1):
        cid = lax.axis_index("c")
        sid = lax.axis_index("s")
        row_base = cid * HALF
        gidx_b = (gidx0, gidx1)
        omap_b = (omap0, omap1)
        rows_b = (rows0, rows1)
        sem_b = (sem0, sem1)

        pltpu.sync_copy(bias_hbm, bias_v.at[pl.ds(0, 1)])
        bvals = [bias_v[0, pl.ds(q * 16, 16)] for q in range(C // 16)]
        for r in range(1, CHUNK):
            for q in range(C // 16):
                bias_v[r, pl.ds(q * 16, 16)] = bvals[q]
        init_base = sid * INIT_ROWS_PER_TILE
        for i in range(INIT_ROWS_PER_TILE // CHUNK):
            pltpu.sync_copy(bias_v, acc.at[pl.ds(init_base + i * CHUNK, CHUNK)])
        rem = INIT_ROWS_PER_TILE % CHUNK
        pltpu.sync_copy(
            bias_v.at[pl.ds(0, rem)],
            acc.at[pl.ds(init_base + INIT_ROWS_PER_TILE - rem, rem)],
        )
        plsc.subcore_barrier()

        def load_and_fire(i, b):
            off = (i * N_SUB + sid) * CHUNK
            pltpu.sync_copy(gidx_hbm.at[pl.ds(off, CHUNK)], gidx_b[b])
            pltpu.sync_copy(omap_hbm.at[pl.ds(off, CHUNK)], omap_b[b])
            pltpu.make_async_copy(
                y_hbm.at[gidx_b[b]], rows_b[b], sem_b[b]
            ).start()

        def drain_and_scatter(b):
            pltpu.make_async_copy(
                y_hbm.at[gidx_b[b]], rows_b[b], sem_b[b]
            ).wait()
            for v in range(CHUNK // 16):
                o = omap_b[b][pl.ds(v * 16, 16)]
                loc = o - row_base
                ok = (loc >= 0) & (loc < HALF)
                idx_v[pl.ds(v * 16, 16)] = jnp.where(ok, loc, TRASH)
            pltpu.sync_copy(rows_b[b], acc.at[idx_v], add=True)

        load_and_fire(0, 0)

        def pair_step(i2, _):
            for b in (0, 1):
                i = i2 * 2 + b

                @pl.when(i + 1 < CHUNKS_PER_TILE)
                def _():
                    load_and_fire(i + 1, 1 - b)

                drain_and_scatter(b)
            return 0

        lax.fori_loop(0, CHUNKS_PER_TILE // 2, pair_step, 0)
        plsc.subcore_barrier()

        for i in range(OUT_FULL_CHUNKS // N_SUB + 1):
            chunk = i * N_SUB + sid
            off = chunk * CHUNK

            @pl.when(chunk < OUT_FULL_CHUNKS)
            def _():
                pltpu.sync_copy(
                    acc.at[pl.ds(off, CHUNK)],
                    out_hbm.at[pl.ds(row_base + off, CHUNK)],
                )

            @pl.when(chunk == OUT_FULL_CHUNKS)
            def _():
                pltpu.sync_copy(
                    acc.at[pl.ds(OUT_FULL_CHUNKS * CHUNK, OUT_TAIL)],
                    out_hbm.at[
                        pl.ds(row_base + OUT_FULL_CHUNKS * CHUNK, OUT_TAIL)
                    ],
                )

    return body(y_flat, gidx, omap_flat, bias_row)


def kernel(in_feats, imap, omap, kernel, bias):
    imap = imap.astype(jnp.int32)
    omap = omap.astype(jnp.int32)

    in2 = in_feats.reshape(N_VOX // 2, 2 * C)
    y128 = _dense_transform(in2, kernel)
    y_flat = y128.reshape(K_VOL * N_VOX, C)

    k_off = (jnp.arange(K_VOL, dtype=jnp.int32) * N_VOX)[:, None]
    gidx = (imap + k_off).reshape(-1)
    pad = N_PAIRS_PAD - N_PAIRS
    gidx = jnp.concatenate([gidx, jnp.zeros((pad,), jnp.int32)])
    omap_flat = jnp.concatenate(
        [omap.reshape(-1), jnp.full((pad,), N_VOX, jnp.int32)]
    )
    return _sc_scatter(y_flat, gidx, omap_flat, bias.reshape(1, C))
